# baseline (speedup 1.0000x reference)
import functools

import jax
import jax.numpy as jnp
from jax import lax
from jax.experimental import pallas as pl
from jax.experimental.pallas import tpu as pltpu
from jax.experimental.pallas import tpu_sc as plsc

S2 = 2048
D = 1024
E = 64
TOPK = 8
NG = 8
TG = 4
DFF = 512
RSF = 2.5
SDFF = 1024

NA = S2 * TOPK
TM = 128
NT = NA // TM + E
NROWS = NT * TM
NW = 32
APW = NA // NW
CH = 64
NCH = APW // CH

_NEG = -1e30


def _pack_halves(lo_bf, hi_bf):
    lo = jax.lax.bitcast_convert_type(lo_bf, jnp.uint16).astype(jnp.uint32)
    hi = jax.lax.bitcast_convert_type(hi_bf, jnp.uint16).astype(jnp.uint32)
    return jax.lax.bitcast_convert_type(lo | (hi << 16), jnp.int32)


def _unpack_halves(xi):
    u = jax.lax.bitcast_convert_type(xi, jnp.uint32)
    lo = jax.lax.bitcast_convert_type((u & 0xFFFF).astype(jnp.uint16),
                                      jnp.bfloat16)
    hi = jax.lax.bitcast_convert_type((u >> 16).astype(jnp.uint16),
                                      jnp.bfloat16)
    return lo, hi


def _route_body(x_ref, gw_ref, b_ref, p8_ref, w8_ref, eot_ref, xi_ref, rank_s):
    x = x_ref[...]
    gw = gw_ref[...]
    logits = jax.lax.dot_general(
        x.astype(jnp.bfloat16), gw.astype(jnp.bfloat16),
        (((1,), (1,)), ((), ())),
        preferred_element_type=jnp.float32,
    )
    scores = jax.nn.sigmoid(logits)
    sc = scores + b_ref[...]
    cols = jax.lax.broadcasted_iota(jnp.int32, (S2, E), 1)
    grp = cols // (E // NG)

    gs_full = jnp.zeros_like(sc)
    for g in range(NG):
        ing = grp == g
        vals = jnp.where(ing, sc, _NEG)
        m1 = jnp.max(vals, axis=-1, keepdims=True)
        i1 = jnp.min(jnp.where(vals == m1, cols, 9999), axis=-1, keepdims=True)
        m2 = jnp.max(jnp.where(cols == i1, _NEG, vals), axis=-1, keepdims=True)
        gs_full = gs_full + jnp.where(ing, m1 + m2, 0.0)

    gsr = gs_full
    chosen = jnp.zeros_like(sc, dtype=jnp.bool_)
    for _ in range(TG):
        m = jnp.max(gsr, axis=-1, keepdims=True)
        gidx = jnp.min(jnp.where(gsr == m, grp, 9999), axis=-1, keepdims=True)
        ch = grp == gidx
        chosen = jnp.logical_or(chosen, ch)
        gsr = jnp.where(ch, _NEG, gsr)

    tmp = jnp.where(chosen, sc, 0.0)
    sel = jnp.zeros_like(sc, dtype=jnp.bool_)
    kcols = jax.lax.broadcasted_iota(jnp.int32, (S2, TOPK), 1)
    iks = []
    w8 = jnp.zeros((S2, TOPK), jnp.float32)
    wsum = jnp.zeros((S2, 1), jnp.float32)
    for k in range(TOPK):
        cur = jnp.where(sel, _NEG, tmp)
        m = jnp.max(cur, axis=-1, keepdims=True)
        ik = jnp.min(jnp.where(cur == m, cols, 9999), axis=-1, keepdims=True)
        sel = jnp.logical_or(sel, cols == ik)
        iks.append(ik)
        wk = jnp.sum(jnp.where(cols == ik, scores, 0.0), axis=-1,
                     keepdims=True)
        w8 = w8 + jnp.where(kcols == k, wk, 0.0)
        wsum = wsum + wk
    w8_ref[...] = w8 * (RSF / (wsum + 1e-20))

    oh = sel.astype(jnp.float32)
    r0 = jax.lax.broadcasted_iota(jnp.int32, (128, 128), 0)
    c0 = jax.lax.broadcasted_iota(jnp.int32, (128, 128), 1)
    ls = (r0 > c0).astype(jnp.float32)
    off = jnp.zeros((1, E), jnp.float32)
    for b in range(S2 // 128):
        ohb = oh[b * 128:(b + 1) * 128, :]
        rb = jax.lax.dot_general(ls, ohb, (((1,), (0,)), ((), ())),
                                 preferred_element_type=jnp.float32)
        rank_s[b * 128:(b + 1) * 128, :] = rb + off
        off = off + jnp.sum(ohb, axis=0, keepdims=True)
    counts = off
    tiles = jnp.floor((counts + (TM - 1)) / TM)
    r64 = jax.lax.broadcasted_iota(jnp.int32, (E, E), 0)
    c64 = jax.lax.broadcasted_iota(jnp.int32, (E, E), 1)
    tex = (r64 < c64).astype(jnp.float32)
    base_t = jax.lax.dot_general(tiles, tex, (((1,), (0,)), ((), ())),
                                 preferred_element_type=jnp.float32)
    p_dense = rank_s[...] + base_t * TM

    p8 = jnp.zeros((S2, TOPK), jnp.float32)
    for k in range(TOPK):
        pk = jnp.sum(jnp.where(cols == iks[k], p_dense, 0.0),
                     axis=-1, keepdims=True)
        p8 = p8 + jnp.where(kcols == k, pk, 0.0)
    p8_ref[...] = p8.astype(jnp.int32)

    jrow = jax.lax.broadcasted_iota(jnp.int32, (1, NT), 1).astype(jnp.float32)
    eotf = jnp.zeros((1, NT), jnp.float32)
    for e in range(E):
        eotf = eotf + (jrow >= base_t[0:1, e:e + 1]).astype(jnp.float32)
    total = jnp.sum(tiles, axis=-1, keepdims=True)
    eot = jnp.where(jrow < total, eotf - 1.0, -1.0)
    eot_ref[...] = eot.astype(jnp.int32)

    xbf = x.astype(jnp.bfloat16)
    xi_ref[...] = _pack_halves(xbf[:, :D // 2], xbf[:, D // 2:])


def _dispatch_body(x_hbm, p_hbm, tok_hbm, wf_hbm, sx_hbm, ws_hbm,
                   tokv0, tokv1, tokv2, posv0, posv1, posv2,
                   wvv0, wvv1, wvv2, rows0, rows1, rows2,
                   semg0, semg1, semg2, sems0, sems1, sems2):
    wid = lax.axis_index("s") * 2 + lax.axis_index("c")
    base = wid * APW
    bufs = [(tokv0, posv0, wvv0, rows0, semg0, sems0),
            (tokv1, posv1, wvv1, rows1, semg1, sems1),
            (tokv2, posv2, wvv2, rows2, semg2, sems2)]

    def load_idx(ch):
        tokv, posv, wvv, _, _, _ = bufs[ch % 3]
        off = base + ch * CH
        pltpu.sync_copy(tok_hbm.at[pl.ds(off, CH)], tokv)
        pltpu.sync_copy(p_hbm.at[pl.ds(off, CH)], posv)
        pltpu.sync_copy(wf_hbm.at[pl.ds(off, CH)], wvv)

    def start_gather(ch):
        tokv, _, _, rows, semg, _ = bufs[ch % 3]
        return pltpu.async_copy(x_hbm.at[tokv], rows, semg)

    def start_scatter(ch):
        _, posv, wvv, rows, _, sems = bufs[ch % 3]
        cs = pltpu.async_copy(rows, sx_hbm.at[posv], sems)
        cw = pltpu.async_copy(wvv, ws_hbm.at[posv], sems)
        return cs, cw

    g = {}
    s = {}
    for pre in range(2):
        load_idx(pre)
        g[pre] = start_gather(pre)
    for ch in range(NCH):
        if ch + 2 < NCH:
            if ch - 1 >= 0:
                s[ch - 1][0].wait()
                s[ch - 1][1].wait()
            load_idx(ch + 2)
            g[ch + 2] = start_gather(ch + 2)
        g[ch].wait()
        s[ch] = start_scatter(ch)
    for ch in (NCH - 3, NCH - 2, NCH - 1):
        s[ch][0].wait()
        s[ch][1].wait()


def _combine_gather_body(os_hbm, p_hbm, ot_hbm,
                         pidx0, pidx1, pidx2, rows0, rows1, rows2,
                         semg0, semg1, semg2, sems0, sems1, sems2):
    wid = lax.axis_index("s") * 2 + lax.axis_index("c")
    base = wid * APW
    bufs = [(pidx0, rows0, semg0, sems0), (pidx1, rows1, semg1, sems1),
            (pidx2, rows2, semg2, sems2)]

    def load_idx(ch):
        pidx, _, _, _ = bufs[ch % 3]
        pltpu.sync_copy(p_hbm.at[pl.ds(base + ch * CH, CH)], pidx)

    def start_gather(ch):
        pidx, rows, semg, _ = bufs[ch % 3]
        return pltpu.async_copy(os_hbm.at[pidx], rows, semg)

    def start_store(ch):
        _, rows, _, sems = bufs[ch % 3]
        return pltpu.async_copy(rows, ot_hbm.at[pl.ds(base + ch * CH, CH)],
                                sems)

    g = {}
    s = {}
    for pre in range(2):
        load_idx(pre)
        g[pre] = start_gather(pre)
    for ch in range(NCH):
        if ch + 2 < NCH:
            if ch - 1 >= 0:
                s[ch - 1].wait()
            load_idx(ch + 2)
            g[ch + 2] = start_gather(ch + 2)
        g[ch].wait()
        s[ch] = start_store(ch)
    for ch in (NCH - 3, NCH - 2, NCH - 1):
        s[ch].wait()


def _gemm_body(eot_s, xs_ref, w_ref, g_ref, u_ref, d_ref, o_ref):
    i = pl.program_id(0)

    @pl.when(eot_s[i] >= 0)
    def _():
        xlo, xhi = _unpack_halves(xs_ref[...])
        g = g_ref[0].astype(jnp.bfloat16)
        u = u_ref[0].astype(jnp.bfloat16)
        a = (jax.lax.dot_general(xlo, g[:, :D // 2], (((1,), (1,)), ((), ())),
                                 preferred_element_type=jnp.float32)
             + jax.lax.dot_general(xhi, g[:, D // 2:],
                                   (((1,), (1,)), ((), ())),
                                   preferred_element_type=jnp.float32))
        b = (jax.lax.dot_general(xlo, u[:, :D // 2], (((1,), (1,)), ((), ())),
                                 preferred_element_type=jnp.float32)
             + jax.lax.dot_general(xhi, u[:, D // 2:],
                                   (((1,), (1,)), ((), ())),
                                   preferred_element_type=jnp.float32))
        h = (a * jax.nn.sigmoid(a)) * b
        d = d_ref[0].astype(jnp.bfloat16)
        o = jax.lax.dot_general(h.astype(jnp.bfloat16), d,
                                (((1,), (1,)), ((), ())),
                                preferred_element_type=jnp.float32)
        ob = (o * w_ref[...]).astype(jnp.bfloat16)
        o_ref[...] = _pack_halves(ob[:, :D // 2], ob[:, D // 2:])


def _reduce_body(ot_ref, xi_ref, sg_ref, su_ref, sd_ref, out_ref):
    xlo, xhi = _unpack_halves(xi_ref[...])
    sg = sg_ref[...].astype(jnp.bfloat16)
    su = su_ref[...].astype(jnp.bfloat16)
    a = (jax.lax.dot_general(xlo, sg[:, :D // 2], (((1,), (1,)), ((), ())),
                             preferred_element_type=jnp.float32)
         + jax.lax.dot_general(xhi, sg[:, D // 2:], (((1,), (1,)), ((), ())),
                               preferred_element_type=jnp.float32))
    b = (jax.lax.dot_general(xlo, su[:, :D // 2], (((1,), (1,)), ((), ())),
                             preferred_element_type=jnp.float32)
         + jax.lax.dot_general(xhi, su[:, D // 2:], (((1,), (1,)), ((), ())),
                               preferred_element_type=jnp.float32))
    h = (a * jax.nn.sigmoid(a)) * b
    sh = jax.lax.dot_general(h.astype(jnp.bfloat16),
                             sd_ref[...].astype(jnp.bfloat16),
                             (((1,), (1,)), ((), ())),
                             preferred_element_type=jnp.float32)
    lo, hi = _unpack_halves(ot_ref[...])
    s_lo = jnp.sum(lo.astype(jnp.float32), axis=1)
    s_hi = jnp.sum(hi.astype(jnp.float32), axis=1)
    out_ref[:, :D // 2] = s_lo + sh[:, :D // 2]
    out_ref[:, D // 2:] = s_hi + sh[:, D // 2:]


def _expert_of(eot_ref, i):
    e = eot_ref[i]
    return jnp.where(e < 0, E - 1, e)


def kernel(hidden_states, gate_weight, e_score_correction_bias, gate_proj,
           up_proj, down_proj, shared_gate, shared_up, shared_down):
    x = hidden_states.reshape(S2, D).astype(jnp.float32)
    bias2d = e_score_correction_bias.reshape(1, E)

    p8, w8, eot, xi = pl.pallas_call(
        _route_body,
        out_shape=(jax.ShapeDtypeStruct((S2, TOPK), jnp.int32),
                   jax.ShapeDtypeStruct((S2, TOPK), jnp.float32),
                   jax.ShapeDtypeStruct((1, NT), jnp.int32),
                   jax.ShapeDtypeStruct((S2, D // 2), jnp.int32)),
        scratch_shapes=[pltpu.VMEM((S2, E), jnp.float32)],
    )(x, gate_weight, bias2d)

    p_flat = p8.reshape(NA)
    w_flat = w8.reshape(NA)
    tok_flat = (jnp.arange(NA, dtype=jnp.int32) // TOPK).astype(jnp.int32)

    mesh = plsc.VectorSubcoreMesh(core_axis_name="c", subcore_axis_name="s",
                                  num_cores=2, num_subcores=16)

    sorted_xi, w_sorted = pl.kernel(
        _dispatch_body,
        out_type=(jax.ShapeDtypeStruct((NROWS, D // 2), jnp.int32),
                  jax.ShapeDtypeStruct((NROWS,), jnp.float32)),
        mesh=mesh,
        scratch_types=(
            [pltpu.VMEM((CH,), jnp.int32)] * 6
            + [pltpu.VMEM((CH,), jnp.float32)] * 3
            + [pltpu.VMEM((CH, D // 2), jnp.int32)] * 3
            + [pltpu.SemaphoreType.DMA] * 6
        ),
    )(xi, p_flat, tok_flat, w_flat)

    out_sorted_i = pl.pallas_call(
        _gemm_body,
        grid_spec=pltpu.PrefetchScalarGridSpec(
            num_scalar_prefetch=1,
            grid=(NT,),
            in_specs=[
                pl.BlockSpec((TM, D // 2), lambda i, eot: (i, 0)),
                pl.BlockSpec((TM, 1), lambda i, eot: (i, 0)),
                pl.BlockSpec((1, DFF, D),
                             lambda i, eot: (_expert_of(eot, i), 0, 0)),
                pl.BlockSpec((1, DFF, D),
                             lambda i, eot: (_expert_of(eot, i), 0, 0)),
                pl.BlockSpec((1, D, DFF),
                             lambda i, eot: (_expert_of(eot, i), 0, 0)),
            ],
            out_specs=pl.BlockSpec((TM, D // 2), lambda i, eot: (i, 0)),
        ),
        out_shape=jax.ShapeDtypeStruct((NROWS, D // 2), jnp.int32),
    )(eot.reshape(NT), sorted_xi,
      w_sorted.reshape(NROWS, 1), gate_proj, up_proj, down_proj)

    out_tok_i = pl.kernel(
        _combine_gather_body,
        out_type=jax.ShapeDtypeStruct((NA, D // 2), jnp.int32),
        mesh=mesh,
        scratch_types=(
            [pltpu.VMEM((CH,), jnp.int32)] * 3
            + [pltpu.VMEM((CH, D // 2), jnp.int32)] * 3
            + [pltpu.SemaphoreType.DMA] * 6
        ),
    )(out_sorted_i, p_flat)

    out = pl.pallas_call(
        _reduce_body,
        grid=(8,),
        in_specs=[
            pl.BlockSpec((S2 // 8, TOPK, D // 2), lambda i: (i, 0, 0)),
            pl.BlockSpec((S2 // 8, D // 2), lambda i: (i, 0)),
            pl.BlockSpec((SDFF, D), lambda i: (0, 0)),
            pl.BlockSpec((SDFF, D), lambda i: (0, 0)),
            pl.BlockSpec((D, SDFF), lambda i: (0, 0)),
        ],
        out_specs=pl.BlockSpec((S2 // 8, D), lambda i: (i, 0)),
        out_shape=jax.ShapeDtypeStruct((S2, D), jnp.float32),
    )(out_tok_i.reshape(S2, TOPK, D // 2), xi,
      shared_gate, shared_up, shared_down)

    return out.reshape(1, S2, D)


# speedup vs baseline: 1.2936x; 1.2936x over previous
import functools

import jax
import jax.numpy as jnp
from jax import lax
from jax.experimental import pallas as pl
from jax.experimental.pallas import tpu as pltpu
from jax.experimental.pallas import tpu_sc as plsc

S2 = 2048
D = 1024
E = 64
TOPK = 8
NG = 8
TG = 4
DFF = 512
RSF = 2.5
SDFF = 1024

NA = S2 * TOPK
TM = 256
NT = NA // TM + E
NROWS = NT * TM
NW = 32
APW = NA // NW
CH = 64
NCH = APW // CH

_NEG = -1e30


def _pack_halves(lo_bf, hi_bf):
    lo = jax.lax.bitcast_convert_type(lo_bf, jnp.uint16).astype(jnp.uint32)
    hi = jax.lax.bitcast_convert_type(hi_bf, jnp.uint16).astype(jnp.uint32)
    return jax.lax.bitcast_convert_type(lo | (hi << 16), jnp.int32)


def _unpack_halves(xi):
    u = jax.lax.bitcast_convert_type(xi, jnp.uint32)
    lo = jax.lax.bitcast_convert_type((u & 0xFFFF).astype(jnp.uint16),
                                      jnp.bfloat16)
    hi = jax.lax.bitcast_convert_type((u >> 16).astype(jnp.uint16),
                                      jnp.bfloat16)
    return lo, hi


def _route_body(x_ref, gw_ref, b_ref, p8_ref, w8_ref, eot_ref, xi_ref, rank_s):
    x = x_ref[...]
    gw = gw_ref[...]
    logits = jax.lax.dot_general(
        x.astype(jnp.bfloat16), gw.astype(jnp.bfloat16),
        (((1,), (1,)), ((), ())),
        preferred_element_type=jnp.float32,
    )
    scores = jax.nn.sigmoid(logits)
    sc = scores + b_ref[...]
    cols = jax.lax.broadcasted_iota(jnp.int32, (S2, E), 1)
    grp = cols // (E // NG)

    gs_full = jnp.zeros_like(sc)
    for g in range(NG):
        ing = grp == g
        vals = jnp.where(ing, sc, _NEG)
        m1 = jnp.max(vals, axis=-1, keepdims=True)
        i1 = jnp.min(jnp.where(vals == m1, cols, 9999), axis=-1, keepdims=True)
        m2 = jnp.max(jnp.where(cols == i1, _NEG, vals), axis=-1, keepdims=True)
        gs_full = gs_full + jnp.where(ing, m1 + m2, 0.0)

    gsr = gs_full
    chosen = jnp.zeros_like(sc, dtype=jnp.bool_)
    for _ in range(TG):
        m = jnp.max(gsr, axis=-1, keepdims=True)
        gidx = jnp.min(jnp.where(gsr == m, grp, 9999), axis=-1, keepdims=True)
        ch = grp == gidx
        chosen = jnp.logical_or(chosen, ch)
        gsr = jnp.where(ch, _NEG, gsr)

    tmp = jnp.where(chosen, sc, 0.0)
    sel = jnp.zeros_like(sc, dtype=jnp.bool_)
    kcols = jax.lax.broadcasted_iota(jnp.int32, (S2, TOPK), 1)
    iks = []
    w8 = jnp.zeros((S2, TOPK), jnp.float32)
    wsum = jnp.zeros((S2, 1), jnp.float32)
    for k in range(TOPK):
        cur = jnp.where(sel, _NEG, tmp)
        m = jnp.max(cur, axis=-1, keepdims=True)
        ik = jnp.min(jnp.where(cur == m, cols, 9999), axis=-1, keepdims=True)
        sel = jnp.logical_or(sel, cols == ik)
        iks.append(ik)
        wk = jnp.sum(jnp.where(cols == ik, scores, 0.0), axis=-1,
                     keepdims=True)
        w8 = w8 + jnp.where(kcols == k, wk, 0.0)
        wsum = wsum + wk
    w8_ref[...] = w8 * (RSF / (wsum + 1e-20))

    oh = sel.astype(jnp.float32)
    r0 = jax.lax.broadcasted_iota(jnp.int32, (128, 128), 0)
    c0 = jax.lax.broadcasted_iota(jnp.int32, (128, 128), 1)
    ls = (r0 > c0).astype(jnp.float32)
    off = jnp.zeros((1, E), jnp.float32)
    for b in range(S2 // 128):
        ohb = oh[b * 128:(b + 1) * 128, :]
        rb = jax.lax.dot_general(ls, ohb, (((1,), (0,)), ((), ())),
                                 preferred_element_type=jnp.float32)
        rank_s[b * 128:(b + 1) * 128, :] = rb + off
        off = off + jnp.sum(ohb, axis=0, keepdims=True)
    counts = off
    tiles = jnp.floor((counts + (TM - 1)) / TM)
    r64 = jax.lax.broadcasted_iota(jnp.int32, (E, E), 0)
    c64 = jax.lax.broadcasted_iota(jnp.int32, (E, E), 1)
    tex = (r64 < c64).astype(jnp.float32)
    base_t = jax.lax.dot_general(tiles, tex, (((1,), (0,)), ((), ())),
                                 preferred_element_type=jnp.float32)
    p_dense = rank_s[...] + base_t * TM

    p8 = jnp.zeros((S2, TOPK), jnp.float32)
    for k in range(TOPK):
        pk = jnp.sum(jnp.where(cols == iks[k], p_dense, 0.0),
                     axis=-1, keepdims=True)
        p8 = p8 + jnp.where(kcols == k, pk, 0.0)
    p8_ref[...] = p8.astype(jnp.int32)

    jrow = jax.lax.broadcasted_iota(jnp.int32, (1, NT), 1).astype(jnp.float32)
    eotf = jnp.zeros((1, NT), jnp.float32)
    for e in range(E):
        eotf = eotf + (jrow >= base_t[0:1, e:e + 1]).astype(jnp.float32)
    total = jnp.sum(tiles, axis=-1, keepdims=True)
    eot = jnp.where(jrow < total, eotf - 1.0, -1.0)
    eot_ref[...] = eot.astype(jnp.int32)

    xbf = x.astype(jnp.bfloat16)
    xi_ref[...] = _pack_halves(xbf[:, :D // 2], xbf[:, D // 2:])


def _dispatch_body(x_hbm, p_hbm, tok_hbm, wf_hbm, sx_hbm, ws_hbm,
                   tokv0, tokv1, tokv2, posv0, posv1, posv2,
                   wvv0, wvv1, wvv2, rows0, rows1, rows2,
                   semg0, semg1, semg2, sems0, sems1, sems2):
    wid = lax.axis_index("s") * 2 + lax.axis_index("c")
    base = wid * APW
    bufs = [(tokv0, posv0, wvv0, rows0, semg0, sems0),
            (tokv1, posv1, wvv1, rows1, semg1, sems1),
            (tokv2, posv2, wvv2, rows2, semg2, sems2)]

    def load_idx(ch):
        tokv, posv, wvv, _, _, _ = bufs[ch % 3]
        off = base + ch * CH
        pltpu.sync_copy(tok_hbm.at[pl.ds(off, CH)], tokv)
        pltpu.sync_copy(p_hbm.at[pl.ds(off, CH)], posv)
        pltpu.sync_copy(wf_hbm.at[pl.ds(off, CH)], wvv)

    def start_gather(ch):
        tokv, _, _, rows, semg, _ = bufs[ch % 3]
        return pltpu.async_copy(x_hbm.at[tokv], rows, semg)

    def start_scatter(ch):
        _, posv, wvv, rows, _, sems = bufs[ch % 3]
        cs = pltpu.async_copy(rows, sx_hbm.at[posv], sems)
        cw = pltpu.async_copy(wvv, ws_hbm.at[posv], sems)
        return cs, cw

    g = {}
    s = {}
    for pre in range(2):
        load_idx(pre)
        g[pre] = start_gather(pre)
    for ch in range(NCH):
        if ch + 2 < NCH:
            if ch - 1 >= 0:
                s[ch - 1][0].wait()
                s[ch - 1][1].wait()
            load_idx(ch + 2)
            g[ch + 2] = start_gather(ch + 2)
        g[ch].wait()
        s[ch] = start_scatter(ch)
    for ch in (NCH - 3, NCH - 2, NCH - 1):
        s[ch][0].wait()
        s[ch][1].wait()


def _combine_gather_body(os_hbm, p_hbm, ot_hbm,
                         pidx0, pidx1, pidx2, rows0, rows1, rows2,
                         semg0, semg1, semg2, sems0, sems1, sems2):
    wid = lax.axis_index("s") * 2 + lax.axis_index("c")
    base = wid * APW
    bufs = [(pidx0, rows0, semg0, sems0), (pidx1, rows1, semg1, sems1),
            (pidx2, rows2, semg2, sems2)]

    def load_idx(ch):
        pidx, _, _, _ = bufs[ch % 3]
        pltpu.sync_copy(p_hbm.at[pl.ds(base + ch * CH, CH)], pidx)

    def start_gather(ch):
        pidx, rows, semg, _ = bufs[ch % 3]
        return pltpu.async_copy(os_hbm.at[pidx], rows, semg)

    def start_store(ch):
        _, rows, _, sems = bufs[ch % 3]
        return pltpu.async_copy(rows, ot_hbm.at[pl.ds(base + ch * CH, CH)],
                                sems)

    g = {}
    s = {}
    for pre in range(2):
        load_idx(pre)
        g[pre] = start_gather(pre)
    for ch in range(NCH):
        if ch + 2 < NCH:
            if ch - 1 >= 0:
                s[ch - 1].wait()
            load_idx(ch + 2)
            g[ch + 2] = start_gather(ch + 2)
        g[ch].wait()
        s[ch] = start_store(ch)
    for ch in (NCH - 3, NCH - 2, NCH - 1):
        s[ch].wait()


def _gemm_body(eot_s, xs_ref, w_ref, g_ref, u_ref, d_ref, o_ref):
    i = pl.program_id(0)

    @pl.when(eot_s[i] >= 0)
    def _():
        xlo, xhi = _unpack_halves(xs_ref[...])
        g = g_ref[0].astype(jnp.bfloat16)
        u = u_ref[0].astype(jnp.bfloat16)
        a = (jax.lax.dot_general(xlo, g[:, :D // 2], (((1,), (1,)), ((), ())),
                                 preferred_element_type=jnp.float32)
             + jax.lax.dot_general(xhi, g[:, D // 2:],
                                   (((1,), (1,)), ((), ())),
                                   preferred_element_type=jnp.float32))
        b = (jax.lax.dot_general(xlo, u[:, :D // 2], (((1,), (1,)), ((), ())),
                                 preferred_element_type=jnp.float32)
             + jax.lax.dot_general(xhi, u[:, D // 2:],
                                   (((1,), (1,)), ((), ())),
                                   preferred_element_type=jnp.float32))
        h = (a * jax.nn.sigmoid(a)) * b
        d = d_ref[0].astype(jnp.bfloat16)
        o = jax.lax.dot_general(h.astype(jnp.bfloat16), d,
                                (((1,), (1,)), ((), ())),
                                preferred_element_type=jnp.float32)
        ob = (o * w_ref[...]).astype(jnp.bfloat16)
        o_ref[...] = _pack_halves(ob[:, :D // 2], ob[:, D // 2:])


def _shared_body(xi_ref, sg_ref, su_ref, sd_ref, out_ref):
    xlo, xhi = _unpack_halves(xi_ref[...])
    sg = sg_ref[...].astype(jnp.bfloat16)
    su = su_ref[...].astype(jnp.bfloat16)
    a = (jax.lax.dot_general(xlo, sg[:, :D // 2], (((1,), (1,)), ((), ())),
                             preferred_element_type=jnp.float32)
         + jax.lax.dot_general(xhi, sg[:, D // 2:], (((1,), (1,)), ((), ())),
                               preferred_element_type=jnp.float32))
    b = (jax.lax.dot_general(xlo, su[:, :D // 2], (((1,), (1,)), ((), ())),
                             preferred_element_type=jnp.float32)
         + jax.lax.dot_general(xhi, su[:, D // 2:], (((1,), (1,)), ((), ())),
                               preferred_element_type=jnp.float32))
    h = (a * jax.nn.sigmoid(a)) * b
    out_ref[...] = jax.lax.dot_general(h.astype(jnp.bfloat16),
                                       sd_ref[...].astype(jnp.bfloat16),
                                       (((1,), (1,)), ((), ())),
                                       preferred_element_type=jnp.float32)


def _reduce_body(ot_ref, sh_ref, out_ref):
    lo, hi = _unpack_halves(ot_ref[...])
    s_lo = jnp.sum(lo.astype(jnp.float32), axis=1)
    s_hi = jnp.sum(hi.astype(jnp.float32), axis=1)
    sh = sh_ref[...]
    out_ref[:, :D // 2] = s_lo + sh[:, :D // 2]
    out_ref[:, D // 2:] = s_hi + sh[:, D // 2:]


def _expert_of(eot_ref, i):
    e = eot_ref[i]
    return jnp.where(e < 0, E - 1, e)


def kernel(hidden_states, gate_weight, e_score_correction_bias, gate_proj,
           up_proj, down_proj, shared_gate, shared_up, shared_down):
    x = hidden_states.reshape(S2, D).astype(jnp.float32)
    bias2d = e_score_correction_bias.reshape(1, E)

    p8, w8, eot, xi = pl.pallas_call(
        _route_body,
        out_shape=(jax.ShapeDtypeStruct((S2, TOPK), jnp.int32),
                   jax.ShapeDtypeStruct((S2, TOPK), jnp.float32),
                   jax.ShapeDtypeStruct((1, NT), jnp.int32),
                   jax.ShapeDtypeStruct((S2, D // 2), jnp.int32)),
        scratch_shapes=[pltpu.VMEM((S2, E), jnp.float32)],
    )(x, gate_weight, bias2d)

    p_flat = p8.reshape(NA)
    w_flat = w8.reshape(NA)
    tok_flat = (jnp.arange(NA, dtype=jnp.int32) // TOPK).astype(jnp.int32)

    mesh = plsc.VectorSubcoreMesh(core_axis_name="c", subcore_axis_name="s",
                                  num_cores=2, num_subcores=16)

    sorted_xi, w_sorted = pl.kernel(
        _dispatch_body,
        out_type=(jax.ShapeDtypeStruct((NROWS, D // 2), jnp.int32),
                  jax.ShapeDtypeStruct((NROWS,), jnp.float32)),
        mesh=mesh,
        scratch_types=(
            [pltpu.VMEM((CH,), jnp.int32)] * 6
            + [pltpu.VMEM((CH,), jnp.float32)] * 3
            + [pltpu.VMEM((CH, D // 2), jnp.int32)] * 3
            + [pltpu.SemaphoreType.DMA] * 6
        ),
    )(xi, p_flat, tok_flat, w_flat)

    shared_out = pl.pallas_call(
        _shared_body,
        grid=(8,),
        in_specs=[
            pl.BlockSpec((S2 // 8, D // 2), lambda i: (i, 0)),
            pl.BlockSpec((SDFF, D), lambda i: (0, 0)),
            pl.BlockSpec((SDFF, D), lambda i: (0, 0)),
            pl.BlockSpec((D, SDFF), lambda i: (0, 0)),
        ],
        out_specs=pl.BlockSpec((S2 // 8, D), lambda i: (i, 0)),
        out_shape=jax.ShapeDtypeStruct((S2, D), jnp.float32),
    )(xi, shared_gate, shared_up, shared_down)

    out_sorted_i = pl.pallas_call(
        _gemm_body,
        grid_spec=pltpu.PrefetchScalarGridSpec(
            num_scalar_prefetch=1,
            grid=(NT,),
            in_specs=[
                pl.BlockSpec((TM, D // 2), lambda i, eot: (i, 0)),
                pl.BlockSpec((TM, 1), lambda i, eot: (i, 0)),
                pl.BlockSpec((1, DFF, D),
                             lambda i, eot: (_expert_of(eot, i), 0, 0)),
                pl.BlockSpec((1, DFF, D),
                             lambda i, eot: (_expert_of(eot, i), 0, 0)),
                pl.BlockSpec((1, D, DFF),
                             lambda i, eot: (_expert_of(eot, i), 0, 0)),
            ],
            out_specs=pl.BlockSpec((TM, D // 2), lambda i, eot: (i, 0)),
        ),
        out_shape=jax.ShapeDtypeStruct((NROWS, D // 2), jnp.int32),
    )(eot.reshape(NT), sorted_xi,
      w_sorted.reshape(NROWS, 1), gate_proj, up_proj, down_proj)

    out_tok_i = pl.kernel(
        _combine_gather_body,
        out_type=jax.ShapeDtypeStruct((NA, D // 2), jnp.int32),
        mesh=mesh,
        scratch_types=(
            [pltpu.VMEM((CH,), jnp.int32)] * 3
            + [pltpu.VMEM((CH, D // 2), jnp.int32)] * 3
            + [pltpu.SemaphoreType.DMA] * 6
        ),
    )(out_sorted_i, p_flat)

    out = pl.pallas_call(
        _reduce_body,
        grid=(8,),
        in_specs=[
            pl.BlockSpec((S2 // 8, TOPK, D // 2), lambda i: (i, 0, 0)),
            pl.BlockSpec((S2 // 8, D), lambda i: (i, 0)),
        ],
        out_specs=pl.BlockSpec((S2 // 8, D), lambda i: (i, 0)),
        out_shape=jax.ShapeDtypeStruct((S2, D), jnp.float32),
    )(out_tok_i.reshape(S2, TOPK, D // 2), shared_out)

    return out.reshape(1, S2, D)


# speedup vs baseline: 1.3013x; 1.0060x over previous
import jax
import jax.numpy as jnp
from jax import lax
from jax.experimental import pallas as pl
from jax.experimental.pallas import tpu as pltpu
from jax.experimental.pallas import tpu_sc as plsc

S2 = 2048
D = 1024
E = 64
TOPK = 8
NG = 8
TG = 4
DFF = 512
RSF = 2.5
SDFF = 1024

NA = S2 * TOPK
TM = 256
NT = NA // TM + E
NROWS = NT * TM
NW = 32
APW = NA // NW
CH = 64
NCH = APW // CH

_NEG = -1e30


def _pack_halves(lo_bf, hi_bf):
    lo = jax.lax.bitcast_convert_type(lo_bf, jnp.uint16).astype(jnp.uint32)
    hi = jax.lax.bitcast_convert_type(hi_bf, jnp.uint16).astype(jnp.uint32)
    return jax.lax.bitcast_convert_type(lo | (hi << 16), jnp.int32)


def _unpack_halves(xi):
    u = jax.lax.bitcast_convert_type(xi, jnp.uint32)
    lo = jax.lax.bitcast_convert_type((u & 0xFFFF).astype(jnp.uint16),
                                      jnp.bfloat16)
    hi = jax.lax.bitcast_convert_type((u >> 16).astype(jnp.uint16),
                                      jnp.bfloat16)
    return lo, hi


def _route_body(x_ref, gw_ref, b_ref, p8_ref, w8_ref, eot_ref, xi_ref, rank_s):
    x = x_ref[...]
    gw = gw_ref[...]
    logits = jax.lax.dot_general(
        x.astype(jnp.bfloat16), gw.astype(jnp.bfloat16),
        (((1,), (1,)), ((), ())),
        preferred_element_type=jnp.float32,
    )
    scores = jax.nn.sigmoid(logits)
    sc = scores + b_ref[...]
    cols = jax.lax.broadcasted_iota(jnp.int32, (S2, E), 1)
    grp = cols // (E // NG)

    gs_full = jnp.zeros_like(sc)
    for g in range(NG):
        ing = grp == g
        vals = jnp.where(ing, sc, _NEG)
        m1 = jnp.max(vals, axis=-1, keepdims=True)
        i1 = jnp.min(jnp.where(vals == m1, cols, 9999), axis=-1, keepdims=True)
        m2 = jnp.max(jnp.where(cols == i1, _NEG, vals), axis=-1, keepdims=True)
        gs_full = gs_full + jnp.where(ing, m1 + m2, 0.0)

    gsr = gs_full
    chosen = jnp.zeros_like(sc, dtype=jnp.bool_)
    for _ in range(TG):
        m = jnp.max(gsr, axis=-1, keepdims=True)
        gidx = jnp.min(jnp.where(gsr == m, grp, 9999), axis=-1, keepdims=True)
        ch = grp == gidx
        chosen = jnp.logical_or(chosen, ch)
        gsr = jnp.where(ch, _NEG, gsr)

    tmp = jnp.where(chosen, sc, 0.0)
    sel = jnp.zeros_like(sc, dtype=jnp.bool_)
    kcols = jax.lax.broadcasted_iota(jnp.int32, (S2, TOPK), 1)
    iks = []
    w8 = jnp.zeros((S2, TOPK), jnp.float32)
    wsum = jnp.zeros((S2, 1), jnp.float32)
    for k in range(TOPK):
        cur = jnp.where(sel, _NEG, tmp)
        m = jnp.max(cur, axis=-1, keepdims=True)
        ik = jnp.min(jnp.where(cur == m, cols, 9999), axis=-1, keepdims=True)
        sel = jnp.logical_or(sel, cols == ik)
        iks.append(ik)
        wk = jnp.sum(jnp.where(cols == ik, scores, 0.0), axis=-1,
                     keepdims=True)
        w8 = w8 + jnp.where(kcols == k, wk, 0.0)
        wsum = wsum + wk
    w8_ref[...] = w8 * (RSF / (wsum + 1e-20))

    oh = sel.astype(jnp.float32)
    r0 = jax.lax.broadcasted_iota(jnp.int32, (128, 128), 0)
    c0 = jax.lax.broadcasted_iota(jnp.int32, (128, 128), 1)
    ls = (r0 > c0).astype(jnp.float32)
    off = jnp.zeros((1, E), jnp.float32)
    for b in range(S2 // 128):
        ohb = oh[b * 128:(b + 1) * 128, :]
        rb = jax.lax.dot_general(ls, ohb, (((1,), (0,)), ((), ())),
                                 preferred_element_type=jnp.float32)
        rank_s[b * 128:(b + 1) * 128, :] = rb + off
        off = off + jnp.sum(ohb, axis=0, keepdims=True)
    counts = off
    tiles = jnp.floor((counts + (TM - 1)) / TM)
    r64 = jax.lax.broadcasted_iota(jnp.int32, (E, E), 0)
    c64 = jax.lax.broadcasted_iota(jnp.int32, (E, E), 1)
    tex = (r64 < c64).astype(jnp.float32)
    base_t = jax.lax.dot_general(tiles, tex, (((1,), (0,)), ((), ())),
                                 preferred_element_type=jnp.float32)
    p_dense = rank_s[...] + base_t * TM

    p8 = jnp.zeros((S2, TOPK), jnp.float32)
    for k in range(TOPK):
        pk = jnp.sum(jnp.where(cols == iks[k], p_dense, 0.0),
                     axis=-1, keepdims=True)
        p8 = p8 + jnp.where(kcols == k, pk, 0.0)
    p8_ref[...] = p8.astype(jnp.int32)

    jrow = jax.lax.broadcasted_iota(jnp.int32, (1, NT), 1).astype(jnp.float32)
    eotf = jnp.zeros((1, NT), jnp.float32)
    for e in range(E):
        eotf = eotf + (jrow >= base_t[0:1, e:e + 1]).astype(jnp.float32)
    total = jnp.sum(tiles, axis=-1, keepdims=True)
    eot = jnp.where(jrow < total, eotf - 1.0, -1.0)
    eot_ref[...] = eot.astype(jnp.int32)

    xbf = x.astype(jnp.bfloat16)
    xi_ref[...] = _pack_halves(xbf[:, :D // 2], xbf[:, D // 2:])


def _dispatch_body(x_hbm, p_hbm, tok_hbm, wf_hbm, sx_hbm, ws_hbm,
                   tokv0, tokv1, tokv2, posv0, posv1, posv2,
                   wvv0, wvv1, wvv2, rows0, rows1, rows2,
                   semg0, semg1, semg2, sems0, sems1, sems2):
    wid = lax.axis_index("s") * 2 + lax.axis_index("c")
    base = wid * APW
    bufs = [(tokv0, posv0, wvv0, rows0, semg0, sems0),
            (tokv1, posv1, wvv1, rows1, semg1, sems1),
            (tokv2, posv2, wvv2, rows2, semg2, sems2)]

    def load_idx(ch):
        tokv, posv, wvv, _, _, _ = bufs[ch % 3]
        off = base + ch * CH
        pltpu.sync_copy(tok_hbm.at[pl.ds(off, CH)], tokv)
        pltpu.sync_copy(p_hbm.at[pl.ds(off, CH)], posv)
        pltpu.sync_copy(wf_hbm.at[pl.ds(off, CH)], wvv)

    def start_gather(ch):
        tokv, _, _, rows, semg, _ = bufs[ch % 3]
        return pltpu.async_copy(x_hbm.at[tokv], rows, semg)

    def start_scatter(ch):
        _, posv, wvv, rows, _, sems = bufs[ch % 3]
        cs = pltpu.async_copy(rows, sx_hbm.at[posv], sems)
        cw = pltpu.async_copy(wvv, ws_hbm.at[posv], sems)
        return cs, cw

    g = {}
    s = {}
    for pre in range(2):
        load_idx(pre)
        g[pre] = start_gather(pre)
    for ch in range(NCH):
        if ch + 2 < NCH:
            if ch - 1 >= 0:
                s[ch - 1][0].wait()
                s[ch - 1][1].wait()
            load_idx(ch + 2)
            g[ch + 2] = start_gather(ch + 2)
        g[ch].wait()
        s[ch] = start_scatter(ch)
    for ch in (NCH - 3, NCH - 2, NCH - 1):
        s[ch][0].wait()
        s[ch][1].wait()


def _combine_gather_body(os_hbm, p_hbm, ot_hbm,
                         pidx0, pidx1, pidx2, rows0, rows1, rows2,
                         semg0, semg1, semg2, sems0, sems1, sems2):
    wid = lax.axis_index("s") * 2 + lax.axis_index("c")
    base = wid * APW
    bufs = [(pidx0, rows0, semg0, sems0), (pidx1, rows1, semg1, sems1),
            (pidx2, rows2, semg2, sems2)]

    def load_idx(ch):
        pidx, _, _, _ = bufs[ch % 3]
        pltpu.sync_copy(p_hbm.at[pl.ds(base + ch * CH, CH)], pidx)

    def start_gather(ch):
        pidx, rows, semg, _ = bufs[ch % 3]
        return pltpu.async_copy(os_hbm.at[pidx], rows, semg)

    def start_store(ch):
        _, rows, _, sems = bufs[ch % 3]
        return pltpu.async_copy(rows, ot_hbm.at[pl.ds(base + ch * CH, CH)],
                                sems)

    g = {}
    s = {}
    for pre in range(2):
        load_idx(pre)
        g[pre] = start_gather(pre)
    for ch in range(NCH):
        if ch + 2 < NCH:
            if ch - 1 >= 0:
                s[ch - 1].wait()
            load_idx(ch + 2)
            g[ch + 2] = start_gather(ch + 2)
        g[ch].wait()
        s[ch] = start_store(ch)
    for ch in (NCH - 3, NCH - 2, NCH - 1):
        s[ch].wait()


def _gemm_body(eot_s, xs_ref, w_ref, g_ref, u_ref, d_ref, o_ref):
    i = pl.program_id(0)

    @pl.when(eot_s[i] >= 0)
    def _():
        xlo, xhi = _unpack_halves(xs_ref[...])
        g = g_ref[0].astype(jnp.bfloat16)
        u = u_ref[0].astype(jnp.bfloat16)
        a = (jax.lax.dot_general(xlo, g[:, :D // 2], (((1,), (1,)), ((), ())),
                                 preferred_element_type=jnp.float32)
             + jax.lax.dot_general(xhi, g[:, D // 2:],
                                   (((1,), (1,)), ((), ())),
                                   preferred_element_type=jnp.float32))
        b = (jax.lax.dot_general(xlo, u[:, :D // 2], (((1,), (1,)), ((), ())),
                                 preferred_element_type=jnp.float32)
             + jax.lax.dot_general(xhi, u[:, D // 2:],
                                   (((1,), (1,)), ((), ())),
                                   preferred_element_type=jnp.float32))
        h = (a * jax.nn.sigmoid(a)) * b
        d = d_ref[0].astype(jnp.bfloat16)
        o = jax.lax.dot_general(h.astype(jnp.bfloat16), d,
                                (((1,), (1,)), ((), ())),
                                preferred_element_type=jnp.float32)
        ob = (o * w_ref[...]).astype(jnp.bfloat16)
        o_ref[...] = _pack_halves(ob[:, :D // 2], ob[:, D // 2:])


def _shared_body(xi_ref, sg_ref, su_ref, sd_ref, out_ref):
    xlo, xhi = _unpack_halves(xi_ref[...])
    sg = sg_ref[...].astype(jnp.bfloat16)
    su = su_ref[...].astype(jnp.bfloat16)
    a = (jax.lax.dot_general(xlo, sg[:, :D // 2], (((1,), (1,)), ((), ())),
                             preferred_element_type=jnp.float32)
         + jax.lax.dot_general(xhi, sg[:, D // 2:], (((1,), (1,)), ((), ())),
                               preferred_element_type=jnp.float32))
    b = (jax.lax.dot_general(xlo, su[:, :D // 2], (((1,), (1,)), ((), ())),
                             preferred_element_type=jnp.float32)
         + jax.lax.dot_general(xhi, su[:, D // 2:], (((1,), (1,)), ((), ())),
                               preferred_element_type=jnp.float32))
    h = (a * jax.nn.sigmoid(a)) * b
    out_ref[...] = jax.lax.dot_general(h.astype(jnp.bfloat16),
                                       sd_ref[...].astype(jnp.bfloat16),
                                       (((1,), (1,)), ((), ())),
                                       preferred_element_type=jnp.float32)


def _reduce_body(ot_ref, sh_ref, out_ref):
    lo, hi = _unpack_halves(ot_ref[...])
    s_lo = jnp.sum(lo.astype(jnp.float32), axis=1)
    s_hi = jnp.sum(hi.astype(jnp.float32), axis=1)
    sh = sh_ref[...]
    out_ref[:, :D // 2] = s_lo + sh[:, :D // 2]
    out_ref[:, D // 2:] = s_hi + sh[:, D // 2:]


def _expert_of(eot_ref, i):
    e = eot_ref[i]
    return jnp.where(e < 0, E - 1, e)


def kernel(hidden_states, gate_weight, e_score_correction_bias, gate_proj,
           up_proj, down_proj, shared_gate, shared_up, shared_down):
    x = hidden_states.reshape(S2, D).astype(jnp.float32)
    bias2d = e_score_correction_bias.reshape(1, E)

    p8, w8, eot, xi = pl.pallas_call(
        _route_body,
        out_shape=(jax.ShapeDtypeStruct((S2, TOPK), jnp.int32),
                   jax.ShapeDtypeStruct((S2, TOPK), jnp.float32),
                   jax.ShapeDtypeStruct((1, NT), jnp.int32),
                   jax.ShapeDtypeStruct((S2, D // 2), jnp.int32)),
        scratch_shapes=[pltpu.VMEM((S2, E), jnp.float32)],
    )(x, gate_weight, bias2d)

    p_flat = p8.reshape(NA)
    w_flat = w8.reshape(NA)
    tok_flat = (jnp.arange(NA, dtype=jnp.int32) // TOPK).astype(jnp.int32)

    mesh = plsc.VectorSubcoreMesh(core_axis_name="c", subcore_axis_name="s",
                                  num_cores=2, num_subcores=16)

    sorted_xi, w_sorted = pl.kernel(
        _dispatch_body,
        out_type=(jax.ShapeDtypeStruct((NROWS, D // 2), jnp.int32),
                  jax.ShapeDtypeStruct((NROWS,), jnp.float32)),
        mesh=mesh,
        scratch_types=(
            [pltpu.VMEM((CH,), jnp.int32)] * 6
            + [pltpu.VMEM((CH,), jnp.float32)] * 3
            + [pltpu.VMEM((CH, D // 2), jnp.int32)] * 3
            + [pltpu.SemaphoreType.DMA] * 6
        ),
    )(xi, p_flat, tok_flat, w_flat)

    shared_out = pl.pallas_call(
        _shared_body,
        grid=(8,),
        in_specs=[
            pl.BlockSpec((S2 // 8, D // 2), lambda i: (i, 0)),
            pl.BlockSpec((SDFF, D), lambda i: (0, 0)),
            pl.BlockSpec((SDFF, D), lambda i: (0, 0)),
            pl.BlockSpec((D, SDFF), lambda i: (0, 0)),
        ],
        out_specs=pl.BlockSpec((S2 // 8, D), lambda i: (i, 0)),
        out_shape=jax.ShapeDtypeStruct((S2, D), jnp.float32),
    )(xi, shared_gate, shared_up, shared_down)

    out_sorted_i = pl.pallas_call(
        _gemm_body,
        grid_spec=pltpu.PrefetchScalarGridSpec(
            num_scalar_prefetch=1,
            grid=(NT,),
            in_specs=[
                pl.BlockSpec((TM, D // 2), lambda i, eot: (i, 0)),
                pl.BlockSpec((TM, 1), lambda i, eot: (i, 0)),
                pl.BlockSpec((1, DFF, D),
                             lambda i, eot: (_expert_of(eot, i), 0, 0)),
                pl.BlockSpec((1, DFF, D),
                             lambda i, eot: (_expert_of(eot, i), 0, 0)),
                pl.BlockSpec((1, D, DFF),
                             lambda i, eot: (_expert_of(eot, i), 0, 0)),
            ],
            out_specs=pl.BlockSpec((TM, D // 2), lambda i, eot: (i, 0)),
        ),
        out_shape=jax.ShapeDtypeStruct((NROWS, D // 2), jnp.int32),
    )(eot.reshape(NT), sorted_xi,
      w_sorted.reshape(NROWS, 1), gate_proj, up_proj, down_proj)

    out_tok_i = pl.kernel(
        _combine_gather_body,
        out_type=jax.ShapeDtypeStruct((NA, D // 2), jnp.int32),
        mesh=mesh,
        scratch_types=(
            [pltpu.VMEM((CH,), jnp.int32)] * 3
            + [pltpu.VMEM((CH, D // 2), jnp.int32)] * 3
            + [pltpu.SemaphoreType.DMA] * 6
        ),
    )(out_sorted_i, p_flat)

    out = pl.pallas_call(
        _reduce_body,
        grid=(8,),
        in_specs=[
            pl.BlockSpec((S2 // 8, TOPK, D // 2), lambda i: (i, 0, 0)),
            pl.BlockSpec((S2 // 8, D), lambda i: (i, 0)),
        ],
        out_specs=pl.BlockSpec((S2 // 8, D), lambda i: (i, 0)),
        out_shape=jax.ShapeDtypeStruct((S2, D), jnp.float32),
    )(out_tok_i.reshape(S2, TOPK, D // 2), shared_out)

    return out.reshape(1, S2, D)
